# baseline (speedup 1.0000x reference)
import functools

import jax
import jax.numpy as jnp
from jax import lax
from jax.experimental import pallas as pl
from jax.experimental.pallas import tpu as pltpu
from jax.experimental.pallas import tpu_sc as plsc

_N = 100000
_G = 1024
_Z = 89
_TBL = 128
_PADIDX = 96
_NC = 2
_NS = 16
_NW = _NC * _NS
_CW = 128
_ROWS_PER_W = 25
_CHUNK = _ROWS_PER_W * _CW
_LAST_BASE = (_NW - 1) * _CHUNK
_LAST_REAL = _N - _LAST_BASE
_LAST_FULL = _LAST_REAL // _CW
_TAIL = _LAST_REAL - _LAST_FULL * _CW
_LAST_ROWS = _LAST_FULL + 1
_VECS_PER_ROW = _CW // 16


def _sc_body(nt_hbm, bt_hbm, tbl_hbm, out_hbm,
             nt_v, val_v, bt2_v, tbl_v, zero_v, acc_sh,
             sem_nt, sem_bt, sem_tb, sem_sc):
    cid = lax.axis_index("c")
    sid = lax.axis_index("s")
    wid = cid * _NS + sid
    is_last = wid == _NW - 1
    base = wid * _CHUNK

    pltpu.async_copy(tbl_hbm, tbl_v.at[pl.ds(0, _Z)], sem_tb)

    def brow(j, carry):
        pltpu.async_copy(bt_hbm.at[pl.ds(base + j * _CW, _CW)],
                         bt2_v.at[j], sem_bt)
        return carry

    @pl.when(jnp.logical_not(is_last))
    def _():
        pltpu.async_copy(nt_hbm.at[pl.ds(base, _CHUNK)], nt_v, sem_nt)
        lax.fori_loop(0, _ROWS_PER_W, brow, 0)

    @pl.when(is_last)
    def _():
        pltpu.async_copy(nt_hbm.at[pl.ds(_LAST_BASE, _LAST_REAL)],
                         nt_v.at[pl.ds(0, _LAST_REAL)], sem_nt)
        lax.fori_loop(0, _LAST_FULL, brow, 0)
        pltpu.async_copy(bt_hbm.at[pl.ds(_N - _TAIL, _TAIL)],
                         bt2_v.at[_LAST_FULL, pl.ds(0, _TAIL)], sem_bt)

    _ZW = _G // _NS
    for i in range(_ZW // 16):
        zero_v[pl.ds(i * 16, 16)] = jnp.zeros((16,), jnp.float32)
    pltpu.sync_copy(zero_v, acc_sh.at[pl.ds(sid * _ZW, _ZW)])

    plsc.subcore_barrier()

    pltpu.make_async_copy(tbl_hbm, tbl_v.at[pl.ds(0, _Z)], sem_tb).wait()
    tbl_v[pl.ds(_PADIDX, 16)] = jnp.zeros((16,), jnp.float32)

    def bdrain(j, carry):
        pltpu.make_async_copy(bt_hbm.at[pl.ds(0, _CW)], bt2_v.at[0],
                              sem_bt).wait()
        return carry

    @pl.when(jnp.logical_not(is_last))
    def _():
        pltpu.make_async_copy(nt_hbm.at[pl.ds(0, _CHUNK)], nt_v,
                              sem_nt).wait()
        lax.fori_loop(0, _ROWS_PER_W, bdrain, 0)

    @pl.when(is_last)
    def _():
        pltpu.make_async_copy(nt_hbm.at[pl.ds(0, _LAST_REAL)],
                              nt_v.at[pl.ds(0, _LAST_REAL)], sem_nt).wait()
        lax.fori_loop(0, _LAST_FULL, bdrain, 0)
        pltpu.make_async_copy(bt_hbm.at[pl.ds(0, _TAIL)],
                              bt2_v.at[_LAST_FULL, pl.ds(0, _TAIL)],
                              sem_bt).wait()
        for k in range(_TAIL // 16, _VECS_PER_ROW):
            nt_v[pl.ds(_LAST_FULL * _CW + k * 16, 16)] = jnp.full(
                (16,), _PADIDX, jnp.int32)
            bt2_v[_LAST_FULL, pl.ds(k * 16, 16)] = jnp.zeros((16,),
                                                             jnp.int32)

    nrows = jnp.where(is_last, _LAST_ROWS, _ROWS_PER_W)

    def row_step(j, carry):
        for k in range(_VECS_PER_ROW):
            off = j * _CW + k * 16
            idx = nt_v[pl.ds(off, 16)]
            val_v[pl.ds(off, 16)] = plsc.load_gather(tbl_v, [idx])
        pltpu.async_copy(val_v.at[pl.ds(j * _CW, _CW)],
                         acc_sh.at[bt2_v.at[j]], sem_sc, add=True)
        return carry

    lax.fori_loop(0, nrows, row_step, 0)

    def drain_step(j, carry):
        pltpu.make_async_copy(val_v.at[pl.ds(0, _CW)],
                              acc_sh.at[bt2_v.at[0]], sem_sc).wait()
        return carry

    lax.fori_loop(0, nrows, drain_step, 0)

    plsc.subcore_barrier()

    @pl.when(sid == 0)
    def _():
        pltpu.sync_copy(acc_sh, out_hbm.at[cid])


@functools.cache
def _sc_call():
    mesh = plsc.VectorSubcoreMesh(
        core_axis_name="c", subcore_axis_name="s",
        num_cores=_NC, num_subcores=_NS)
    return pl.kernel(
        _sc_body,
        out_type=jax.ShapeDtypeStruct((_NC, _G), jnp.float32),
        mesh=mesh,
        compiler_params=pltpu.CompilerParams(needs_layout_passes=False),
        scratch_types=[
            pltpu.VMEM((_CHUNK,), jnp.int32),
            pltpu.VMEM((_CHUNK,), jnp.float32),
            pltpu.VMEM((_ROWS_PER_W, _CW), jnp.int32),
            pltpu.VMEM((_TBL,), jnp.float32),
            pltpu.VMEM((_G // _NS,), jnp.float32),
            pltpu.VMEM_SHARED((_G,), jnp.float32),
            pltpu.SemaphoreType.DMA,
            pltpu.SemaphoreType.DMA,
            pltpu.SemaphoreType.DMA,
            pltpu.SemaphoreType.DMA,
        ],
    )


def kernel(node_type, batch, property_offset):
    nt = node_type.astype(jnp.int32)
    bt = batch.astype(jnp.int32)
    tbl = property_offset.astype(jnp.float32)
    partial = _sc_call()(nt, bt, tbl)
    return partial[0] + partial[1]


# speedup vs baseline: 1.0285x; 1.0285x over previous
import functools

import jax
import jax.numpy as jnp
from jax import lax
from jax.experimental import pallas as pl
from jax.experimental.pallas import tpu as pltpu
from jax.experimental.pallas import tpu_sc as plsc

_N = 100000
_G = 1024
_Z = 89
_TBL = 128
_PADIDX = 96
_NC = 2
_NS = 16
_NW = _NC * _NS
_CW = 128
_ROWS_PER_W = 25
_CHUNK = _ROWS_PER_W * _CW
_LAST_BASE = (_NW - 1) * _CHUNK
_LAST_REAL = _N - _LAST_BASE
_LAST_FULL = _LAST_REAL // _CW
_TAIL = _LAST_REAL - _LAST_FULL * _CW
_LAST_ROWS = _LAST_FULL + 1
_VECS_PER_ROW = _CW // 16
_S = 8
_ACC = _S * _G
_ZW = _ACC // _NS
_FW = _G // _NS


def _sc_body(nt_hbm, bt_hbm, tbl_hbm, out_hbm,
             nt_v, val_v, bt2_v, tbl_v, zero_v, facc_v, fold_v, acc_sh,
             sem_nt, sem_bt, sem_tb, sem_sc, sem_fd):
    cid = lax.axis_index("c")
    sid = lax.axis_index("s")
    wid = cid * _NS + sid
    is_last = wid == _NW - 1
    base = wid * _CHUNK

    pltpu.async_copy(tbl_hbm, tbl_v.at[pl.ds(0, _Z)], sem_tb)

    def brow(j, carry):
        pltpu.async_copy(bt_hbm.at[pl.ds(base + j * _CW, _CW)],
                         bt2_v.at[j], sem_bt)
        return carry

    @pl.when(jnp.logical_not(is_last))
    def _():
        pltpu.async_copy(nt_hbm.at[pl.ds(base, _CHUNK)], nt_v, sem_nt)
        lax.fori_loop(0, _ROWS_PER_W, brow, 0)

    @pl.when(is_last)
    def _():
        pltpu.async_copy(nt_hbm.at[pl.ds(_LAST_BASE, _LAST_REAL)],
                         nt_v.at[pl.ds(0, _LAST_REAL)], sem_nt)
        lax.fori_loop(0, _LAST_FULL, brow, 0)
        pltpu.async_copy(bt_hbm.at[pl.ds(_N - _TAIL, _TAIL)],
                         bt2_v.at[_LAST_FULL, pl.ds(0, _TAIL)], sem_bt)

    for i in range(_ZW // 16):
        zero_v[pl.ds(i * 16, 16)] = jnp.zeros((16,), jnp.float32)
    pltpu.sync_copy(zero_v, acc_sh.at[pl.ds(sid * _ZW, _ZW)])

    plsc.subcore_barrier()

    pltpu.make_async_copy(tbl_hbm, tbl_v.at[pl.ds(0, _Z)], sem_tb).wait()
    tbl_v[pl.ds(_PADIDX, 16)] = jnp.zeros((16,), jnp.float32)

    def bdrain(j, carry):
        pltpu.make_async_copy(bt_hbm.at[pl.ds(0, _CW)], bt2_v.at[0],
                              sem_bt).wait()
        return carry

    @pl.when(jnp.logical_not(is_last))
    def _():
        pltpu.make_async_copy(nt_hbm.at[pl.ds(0, _CHUNK)], nt_v,
                              sem_nt).wait()
        lax.fori_loop(0, _ROWS_PER_W, bdrain, 0)

    @pl.when(is_last)
    def _():
        pltpu.make_async_copy(nt_hbm.at[pl.ds(0, _LAST_REAL)],
                              nt_v.at[pl.ds(0, _LAST_REAL)], sem_nt).wait()
        lax.fori_loop(0, _LAST_FULL, bdrain, 0)
        pltpu.make_async_copy(bt_hbm.at[pl.ds(0, _TAIL)],
                              bt2_v.at[_LAST_FULL, pl.ds(0, _TAIL)],
                              sem_bt).wait()
        for k in range(_TAIL // 16, _VECS_PER_ROW):
            nt_v[pl.ds(_LAST_FULL * _CW + k * 16, 16)] = jnp.full(
                (16,), _PADIDX, jnp.int32)
            bt2_v[_LAST_FULL, pl.ds(k * 16, 16)] = jnp.zeros((16,),
                                                             jnp.int32)

    nrows = jnp.where(is_last, _LAST_ROWS, _ROWS_PER_W)

    salt = (lax.iota(jnp.int32, 16) % _S) * _G

    def salt_step(j, carry):
        for k in range(_VECS_PER_ROW):
            bt2_v[j, pl.ds(k * 16, 16)] = (
                bt2_v[j, pl.ds(k * 16, 16)] + salt)
        return carry

    lax.fori_loop(0, nrows, salt_step, 0)

    def row_step(j, carry):
        for k in range(_VECS_PER_ROW):
            off = j * _CW + k * 16
            idx = nt_v[pl.ds(off, 16)]
            val_v[pl.ds(off, 16)] = plsc.load_gather(tbl_v, [idx])
        pltpu.async_copy(val_v.at[pl.ds(j * _CW, _CW)],
                         acc_sh.at[bt2_v.at[j]], sem_sc, add=True)
        return carry

    lax.fori_loop(0, nrows, row_step, 0)

    def drain_step(j, carry):
        pltpu.make_async_copy(val_v.at[pl.ds(0, _CW)],
                              acc_sh.at[bt2_v.at[0]], sem_sc).wait()
        return carry

    lax.fori_loop(0, nrows, drain_step, 0)

    plsc.subcore_barrier()

    for c in range(_S):
        pltpu.async_copy(acc_sh.at[pl.ds(c * _G + sid * _FW, _FW)],
                         facc_v.at[pl.ds(c * _FW, _FW)], sem_fd)
    for c in range(_S):
        pltpu.make_async_copy(acc_sh.at[pl.ds(0, _FW)],
                              facc_v.at[pl.ds(0, _FW)], sem_fd).wait()
    for i in range(_FW // 16):
        tot = facc_v[pl.ds(i * 16, 16)]
        for c in range(1, _S):
            tot = tot + facc_v[pl.ds(c * _FW + i * 16, 16)]
        fold_v[pl.ds(i * 16, 16)] = tot
    pltpu.sync_copy(fold_v, out_hbm.at[cid, pl.ds(sid * _FW, _FW)])


@functools.cache
def _sc_call():
    mesh = plsc.VectorSubcoreMesh(
        core_axis_name="c", subcore_axis_name="s",
        num_cores=_NC, num_subcores=_NS)
    return pl.kernel(
        _sc_body,
        out_type=jax.ShapeDtypeStruct((_NC, _G), jnp.float32),
        mesh=mesh,
        compiler_params=pltpu.CompilerParams(needs_layout_passes=False),
        scratch_types=[
            pltpu.VMEM((_CHUNK,), jnp.int32),
            pltpu.VMEM((_CHUNK,), jnp.float32),
            pltpu.VMEM((_ROWS_PER_W, _CW), jnp.int32),
            pltpu.VMEM((_TBL,), jnp.float32),
            pltpu.VMEM((_ZW,), jnp.float32),
            pltpu.VMEM((_S * _FW,), jnp.float32),
            pltpu.VMEM((_FW,), jnp.float32),
            pltpu.VMEM_SHARED((_ACC,), jnp.float32),
            pltpu.SemaphoreType.DMA,
            pltpu.SemaphoreType.DMA,
            pltpu.SemaphoreType.DMA,
            pltpu.SemaphoreType.DMA,
            pltpu.SemaphoreType.DMA,
        ],
    )


def kernel(node_type, batch, property_offset):
    nt = node_type.astype(jnp.int32)
    bt = batch.astype(jnp.int32)
    tbl = property_offset.astype(jnp.float32)
    partial = _sc_call()(nt, bt, tbl)
    return partial[0] + partial[1]
